# baseline (speedup 1.0000x reference)
import functools

import jax
import jax.numpy as jnp
from jax import lax
from jax.experimental import pallas as pl
from jax.experimental.pallas import tpu as pltpu
from jax.experimental.pallas import tpu_sc as plsc

N = 10000
E = 320000
DZ = 128
DH = 32

NC = 2
NS = 16
NW = NC * NS
CH = 80
NCHUNK = 125
NP = 10240
RPT = NP // NS
NSET = 4
UNR = 8

ROWBLK_A = 1000
ROWBLK_C = 1000


def _proj_body(z_ref, wd_ref, ws_ref, bn_ref, ea_ref, eb_ref):
    zb = z_ref[...]
    ea_ref[...] = jnp.exp(
        -(jnp.dot(zb, wd_ref[...], preferred_element_type=jnp.float32)
          + bn_ref[...])
    )
    eb_ref[...] = jnp.exp(
        -jnp.dot(zb, ws_ref[...], preferred_element_type=jnp.float32)
    )


def _proj(z, wd, ws, bn):
    grid = (N // ROWBLK_A,)
    return pl.pallas_call(
        _proj_body,
        grid=grid,
        in_specs=[
            pl.BlockSpec((ROWBLK_A, DZ), lambda i: (i, 0)),
            pl.BlockSpec((DZ, DH), lambda i: (0, 0)),
            pl.BlockSpec((DZ, DH), lambda i: (0, 0)),
            pl.BlockSpec((1, DH), lambda i: (0, 0)),
        ],
        out_specs=[
            pl.BlockSpec((ROWBLK_A, DH), lambda i: (i, 0)),
            pl.BlockSpec((ROWBLK_A, DH), lambda i: (i, 0)),
        ],
        out_shape=[
            jax.ShapeDtypeStruct((NP, DH), jnp.float32),
            jax.ShapeDtypeStruct((NP, DH), jnp.float32),
        ],
    )(z, wd, ws, bn)


_MESH = plsc.VectorSubcoreMesh(core_axis_name="c", subcore_axis_name="s")


@functools.partial(
    pl.kernel,
    out_type=jax.ShapeDtypeStruct((NC, NP, DH), jnp.float32),
    mesh=_MESH,
    scratch_types=[
        pltpu.VMEM((NCHUNK, CH), jnp.int32),
        pltpu.VMEM((NCHUNK, CH), jnp.int32),
        [pltpu.VMEM((CH, DH), jnp.float32)] * NSET,
        [pltpu.VMEM((CH, DH), jnp.bfloat16)] * NSET,
        pltpu.VMEM((RPT, DH), jnp.float32),
        pltpu.VMEM_SHARED((NP, DH), jnp.float32),
        pltpu.VMEM_SHARED((NP, DH), jnp.bfloat16),
        pltpu.VMEM_SHARED((NP, DH), jnp.float32),
        [pltpu.SemaphoreType.DMA] * NSET,
        [pltpu.SemaphoreType.DMA] * NSET,
        [pltpu.SemaphoreType.DMA] * NSET,
    ],
    compiler_params=pltpu.CompilerParams(
        use_tc_tiling_on_sc=False,
        disable_bounds_checks=True,
        needs_layout_passes=False,
    ),
)
def _edge_kernel(ea_hbm, eb_hbm, dst_hbm, src_hbm, out_hbm,
                 dst_v, src_v, ra, rb, stage, ea_sp, eb_sp, acc,
                 sga, sgb, ssc):
    c = lax.axis_index("c")
    s = lax.axis_index("s")
    wid = c * NS + s
    sl = pl.ds(s * RPT, RPT)

    pltpu.sync_copy(dst_hbm.at[wid], dst_v)
    pltpu.sync_copy(src_hbm.at[wid], src_v)
    pltpu.sync_copy(ea_hbm.at[sl], ea_sp.at[sl])
    pltpu.sync_copy(eb_hbm.at[sl], eb_sp.at[sl])

    zero16 = jnp.zeros((16,), jnp.float32)

    def zero_body(i, carry):
        r0 = i * UNR
        for u in range(UNR):
            stage[r0 + u, pl.ds(0, 16)] = zero16
            stage[r0 + u, pl.ds(16, 16)] = zero16
        return carry

    lax.fori_loop(0, RPT // UNR, zero_body, 0)
    pltpu.sync_copy(stage, acc.at[sl])
    plsc.subcore_barrier()

    def issue_gather(j, t):
        pltpu.async_copy(ea_sp.at[dst_v.at[j]], ra[t], sga[t])
        pltpu.async_copy(eb_sp.at[src_v.at[j]], rb[t], sgb[t])

    def wait_gather(j, t):
        pltpu.make_async_copy(ea_sp.at[dst_v.at[j]], ra[t], sga[t]).wait()
        pltpu.make_async_copy(eb_sp.at[src_v.at[j]], rb[t], sgb[t]).wait()

    def issue_scatter(j, t):
        pltpu.async_copy(ra[t], acc.at[dst_v.at[j]], ssc[t], add=True)

    def wait_scatter(j, t):
        pltpu.make_async_copy(ra[t], acc.at[dst_v.at[j]], ssc[t]).wait()

    def compute(t):
        ra_t = ra[t]
        rb_t = rb[t]
        one16 = jnp.full((16,), 1.0, jnp.float32)

        def row_body(i, rc):
            r0 = i * UNR
            for u in range(UNR):
                eb32 = rb_t[r0 + u, pl.ds(0, DH)]
                b_lo, b_hi = plsc.unpack(eb32, format=plsc.PackFormat.INTERLEAVED)
                for h, bb in ((0, b_lo), (16, b_hi)):
                    p = ra_t[r0 + u, pl.ds(h, 16)] * bb
                    ra_t[r0 + u, pl.ds(h, 16)] = one16 / (1.0 + p)
            return rc

        lax.fori_loop(0, CH // UNR, row_body, 0)

    def run_slot(j, t, issue_ahead, wait_behind):
        if wait_behind:
            wait_scatter(j - 2, (t + 2) % NSET)
        if issue_ahead:
            issue_gather(j + 2, (t + 2) % NSET)
        wait_gather(j, t)
        compute(t)
        issue_scatter(j, t)

    issue_gather(0, 0)
    issue_gather(1, 1)
    run_slot(jnp.int32(0), 0, True, False)
    run_slot(jnp.int32(1), 1, True, False)

    def group_body(g, carry):
        j0 = 2 + 4 * g
        run_slot(j0 + 0, 2, True, True)
        run_slot(j0 + 1, 3, True, True)
        run_slot(j0 + 2, 0, True, True)
        run_slot(j0 + 3, 1, True, True)
        return carry

    lax.fori_loop(0, 30, group_body, 0)

    run_slot(jnp.int32(122), 2, True, True)
    run_slot(jnp.int32(123), 3, False, True)
    run_slot(jnp.int32(124), 0, False, True)

    wait_scatter(jnp.int32(NCHUNK - 2), 3)
    wait_scatter(jnp.int32(NCHUNK - 1), 0)

    plsc.subcore_barrier()

    pltpu.sync_copy(acc.at[sl], stage)
    pltpu.sync_copy(stage, out_hbm.at[c, sl])


def _final_body(z_ref, p0_ref, p1_ref, wc_ref, bc_ref, wt_ref, wb_ref, bo_ref,
                out_ref):
    zb = z_ref[...]
    cur = jax.nn.sigmoid(
        jnp.dot(zb, wc_ref[...], preferred_element_type=jnp.float32) + bc_ref[...]
    )
    nbr = p0_ref[0] + p1_ref[0]
    out_ref[...] = jnp.tanh(
        jnp.dot(cur, wt_ref[...], preferred_element_type=jnp.float32)
        + jnp.dot(nbr, wb_ref[...], preferred_element_type=jnp.float32)
        + bo_ref[...]
    )


def _final(z, partial, wc, bc, wt, wb, bo):
    grid = (N // ROWBLK_C,)
    return pl.pallas_call(
        _final_body,
        grid=grid,
        in_specs=[
            pl.BlockSpec((ROWBLK_C, DZ), lambda i: (i, 0)),
            pl.BlockSpec((1, ROWBLK_C, DH), lambda i: (0, i, 0)),
            pl.BlockSpec((1, ROWBLK_C, DH), lambda i: (1, i, 0)),
            pl.BlockSpec((DZ, DH), lambda i: (0, 0)),
            pl.BlockSpec((1, DH), lambda i: (0, 0)),
            pl.BlockSpec((DH, DZ), lambda i: (0, 0)),
            pl.BlockSpec((DH, DZ), lambda i: (0, 0)),
            pl.BlockSpec((1, DZ), lambda i: (0, 0)),
        ],
        out_specs=pl.BlockSpec((ROWBLK_C, DZ), lambda i: (i, 0)),
        out_shape=jax.ShapeDtypeStruct((N, DZ), jnp.float32),
    )(z, partial, partial, wc, bc, wt, wb, bo)


def kernel(z, edge_index, W_cur, b_cur, W_nbr, b_nbr, W_out, b_out):
    wd = W_nbr[:DZ]
    ws = W_nbr[DZ:]
    ea, eb = _proj(z, wd, ws, b_nbr.reshape(1, DH))
    ebi = (
        jnp.stack([eb[:, : DH // 2], eb[:, DH // 2:]], axis=2)
        .reshape(NP, DH)
        .astype(jnp.bfloat16)
    )

    dst2 = edge_index[0].reshape(NW, NCHUNK, CH)
    src2 = edge_index[1].reshape(NW, NCHUNK, CH)
    partial = _edge_kernel(ea, ebi, dst2, src2)

    return _final(
        z, partial,
        W_cur, b_cur.reshape(1, DH),
        W_out[:DH], W_out[DH:], b_out.reshape(1, DZ),
    )


# speedup vs baseline: 1.7758x; 1.7758x over previous
import functools

import jax
import jax.numpy as jnp
from jax import lax
from jax.experimental import pallas as pl
from jax.experimental.pallas import tpu as pltpu
from jax.experimental.pallas import tpu_sc as plsc

N = 10000
E = 320000
DZ = 128
DH = 32

NC = 2
NS = 16
NW = NC * NS
CH = 80
NCHUNK = 125
NP = 10240
RPT = NP // NS
NSET = 4
UNR = 8

ROWBLK_A = 1000
ROWBLK_C = 1000


def _proj_body(z_ref, wd_ref, ws_ref, bn_ref, ea_ref, eb_ref):
    zb = z_ref[...]
    ea_ref[...] = jnp.exp(
        -(jnp.dot(zb, wd_ref[...], preferred_element_type=jnp.float32)
          + bn_ref[...])
    )
    eb_ref[...] = jnp.exp(
        -jnp.dot(zb, ws_ref[...], preferred_element_type=jnp.float32)
    )


def _proj(z, wd, ws, bn):
    grid = (N // ROWBLK_A,)
    return pl.pallas_call(
        _proj_body,
        grid=grid,
        in_specs=[
            pl.BlockSpec((ROWBLK_A, DZ), lambda i: (i, 0)),
            pl.BlockSpec((DZ, DH), lambda i: (0, 0)),
            pl.BlockSpec((DZ, DH), lambda i: (0, 0)),
            pl.BlockSpec((1, DH), lambda i: (0, 0)),
        ],
        out_specs=[
            pl.BlockSpec((ROWBLK_A, DH), lambda i: (i, 0)),
            pl.BlockSpec((ROWBLK_A, DH), lambda i: (i, 0)),
        ],
        out_shape=[
            jax.ShapeDtypeStruct((NP, DH), jnp.float32),
            jax.ShapeDtypeStruct((NP, DH), jnp.float32),
        ],
    )(z, wd, ws, bn)


_MESH = plsc.VectorSubcoreMesh(core_axis_name="c", subcore_axis_name="s")


@functools.partial(
    pl.kernel,
    out_type=jax.ShapeDtypeStruct((NC, NP, DH), jnp.float32),
    mesh=_MESH,
    scratch_types=[
        pltpu.VMEM((NCHUNK, CH), jnp.int32),
        pltpu.VMEM((NCHUNK, CH), jnp.int32),
        [pltpu.VMEM((CH, DH), jnp.float32)] * NSET,
        [pltpu.VMEM((CH, DH), jnp.float32)] * NSET,
        pltpu.VMEM((RPT, DH), jnp.float32),
        pltpu.VMEM_SHARED((NP, DH), jnp.float32),
        pltpu.VMEM_SHARED((NP, DH), jnp.float32),
        pltpu.VMEM_SHARED((NP, DH), jnp.float32),
        [pltpu.SemaphoreType.DMA] * NSET,
        [pltpu.SemaphoreType.DMA] * NSET,
        [pltpu.SemaphoreType.DMA] * NSET,
    ],
    compiler_params=pltpu.CompilerParams(
        use_tc_tiling_on_sc=False,
        disable_bounds_checks=True,
    ),
)
def _edge_kernel(ea_hbm, eb_hbm, dst_hbm, src_hbm, out_hbm,
                 dst_v, src_v, ra, rb, stage, ea_sp, eb_sp, acc,
                 sga, sgb, ssc):
    c = lax.axis_index("c")
    s = lax.axis_index("s")
    wid = c * NS + s
    sl = pl.ds(s * RPT, RPT)

    pltpu.async_copy(dst_hbm.at[wid], dst_v, sga[0])
    pltpu.async_copy(src_hbm.at[wid], src_v, sga[1])
    pltpu.async_copy(ea_hbm.at[sl], ea_sp.at[sl], sga[2])
    pltpu.async_copy(eb_hbm.at[sl], eb_sp.at[sl], sga[3])

    zero16 = jnp.zeros((16,), jnp.float32)

    def zero_body(i, carry):
        r0 = i * UNR
        for u in range(UNR):
            stage[r0 + u, pl.ds(0, 16)] = zero16
            stage[r0 + u, pl.ds(16, 16)] = zero16
        return carry

    lax.fori_loop(0, RPT // UNR, zero_body, 0)
    pltpu.make_async_copy(dst_hbm.at[wid], dst_v, sga[0]).wait()
    pltpu.make_async_copy(src_hbm.at[wid], src_v, sga[1]).wait()
    pltpu.make_async_copy(ea_hbm.at[sl], ea_sp.at[sl], sga[2]).wait()
    pltpu.make_async_copy(eb_hbm.at[sl], eb_sp.at[sl], sga[3]).wait()
    pltpu.sync_copy(stage, acc.at[sl])
    plsc.subcore_barrier()

    def issue_gather(j, t):
        pltpu.async_copy(ea_sp.at[dst_v.at[j]], ra[t], sga[t])
        pltpu.async_copy(eb_sp.at[src_v.at[j]], rb[t], sgb[t])

    def wait_gather(j, t):
        pltpu.make_async_copy(ea_sp.at[dst_v.at[j]], ra[t], sga[t]).wait()
        pltpu.make_async_copy(eb_sp.at[src_v.at[j]], rb[t], sgb[t]).wait()

    def issue_scatter(j, t):
        pltpu.async_copy(ra[t], acc.at[dst_v.at[j]], ssc[t], add=True)

    def wait_scatter(j, t):
        pltpu.make_async_copy(ra[t], acc.at[dst_v.at[j]], ssc[t]).wait()

    def compute(t):
        ra_t = ra[t]
        rb_t = rb[t]
        one16 = jnp.full((16,), 1.0, jnp.float32)

        def row_body(i, rc):
            r0 = i * UNR
            for u in range(UNR):
                for h in (0, 16):
                    p = ra_t[r0 + u, pl.ds(h, 16)] * rb_t[r0 + u, pl.ds(h, 16)]
                    ra_t[r0 + u, pl.ds(h, 16)] = one16 / (1.0 + p)
            return rc

        lax.fori_loop(0, CH // UNR, row_body, 0)

    def run_slot(j, t, issue_ahead, wait_behind):
        if wait_behind:
            wait_scatter(j - 2, (t + 2) % NSET)
        if issue_ahead:
            issue_gather(j + 2, (t + 2) % NSET)
        wait_gather(j, t)
        compute(t)
        issue_scatter(j, t)

    issue_gather(0, 0)
    issue_gather(1, 1)
    run_slot(jnp.int32(0), 0, True, False)
    run_slot(jnp.int32(1), 1, True, False)

    def group_body(g, carry):
        j0 = 2 + 4 * g
        run_slot(j0 + 0, 2, True, True)
        run_slot(j0 + 1, 3, True, True)
        run_slot(j0 + 2, 0, True, True)
        run_slot(j0 + 3, 1, True, True)
        return carry

    lax.fori_loop(0, 30, group_body, 0)

    run_slot(jnp.int32(122), 2, True, True)
    run_slot(jnp.int32(123), 3, False, True)
    run_slot(jnp.int32(124), 0, False, True)

    wait_scatter(jnp.int32(NCHUNK - 2), 3)
    wait_scatter(jnp.int32(NCHUNK - 1), 0)

    plsc.subcore_barrier()

    pltpu.sync_copy(acc.at[sl], stage)
    pltpu.sync_copy(stage, out_hbm.at[c, sl])


def _final_body(z_ref, p0_ref, p1_ref, wc_ref, bc_ref, wt_ref, wb_ref, bo_ref,
                out_ref):
    zb = z_ref[...]
    cur = jax.nn.sigmoid(
        jnp.dot(zb, wc_ref[...], preferred_element_type=jnp.float32) + bc_ref[...]
    )
    nbr = p0_ref[0] + p1_ref[0]
    out_ref[...] = jnp.tanh(
        jnp.dot(cur, wt_ref[...], preferred_element_type=jnp.float32)
        + jnp.dot(nbr, wb_ref[...], preferred_element_type=jnp.float32)
        + bo_ref[...]
    )


def _final(z, partial, wc, bc, wt, wb, bo):
    grid = (N // ROWBLK_C,)
    return pl.pallas_call(
        _final_body,
        grid=grid,
        in_specs=[
            pl.BlockSpec((ROWBLK_C, DZ), lambda i: (i, 0)),
            pl.BlockSpec((1, ROWBLK_C, DH), lambda i: (0, i, 0)),
            pl.BlockSpec((1, ROWBLK_C, DH), lambda i: (1, i, 0)),
            pl.BlockSpec((DZ, DH), lambda i: (0, 0)),
            pl.BlockSpec((1, DH), lambda i: (0, 0)),
            pl.BlockSpec((DH, DZ), lambda i: (0, 0)),
            pl.BlockSpec((DH, DZ), lambda i: (0, 0)),
            pl.BlockSpec((1, DZ), lambda i: (0, 0)),
        ],
        out_specs=pl.BlockSpec((ROWBLK_C, DZ), lambda i: (i, 0)),
        out_shape=jax.ShapeDtypeStruct((N, DZ), jnp.float32),
    )(z, partial, partial, wc, bc, wt, wb, bo)


def kernel(z, edge_index, W_cur, b_cur, W_nbr, b_nbr, W_out, b_out):
    wd = W_nbr[:DZ]
    ws = W_nbr[DZ:]
    ea, eb = _proj(z, wd, ws, b_nbr.reshape(1, DH))

    dst2 = edge_index[0].reshape(NW, NCHUNK, CH)
    src2 = edge_index[1].reshape(NW, NCHUNK, CH)
    partial = _edge_kernel(ea, eb, dst2, src2)

    return _final(
        z, partial,
        W_cur, b_cur.reshape(1, DH),
        W_out[:DH], W_out[DH:], b_out.reshape(1, DZ),
    )
